# baseline (speedup 1.0000x reference)
import functools

import jax
import jax.numpy as jnp
from jax import lax
from jax.experimental import pallas as pl
from jax.experimental.pallas import tpu as pltpu
from jax.experimental.pallas import tpu_sc as plsc

N = 10000
E = 160000
D_IN, H1, H2 = 256, 128, 32

NCORES = 2
NSUB = 16
NWORK = NCORES * NSUB
EPW = E // NWORK
CHUNK = 80
NCHUNK = (E // NSUB) // CHUNK
NBUF = 5
OWN = 632
OWN_LAST = N - 15 * OWN
ZROWS = 40


def _mm1_body(x_ref, w_ref, a0l_ref, a0h_ref, a1l_ref, a1h_ref, s_ref):
    acc = jnp.dot(x_ref[...], w_ref[...], preferred_element_type=jnp.float32)
    hh = H1 // 2
    s = acc[:, 0:H1]
    a0 = s + acc[:, H1:2 * H1]
    a1 = s + acc[:, 2 * H1:3 * H1]
    a0l_ref[...] = a0[:, 0:hh]
    a0h_ref[...] = a0[:, hh:2 * hh]
    a1l_ref[...] = a1[:, 0:hh]
    a1h_ref[...] = a1[:, hh:2 * hh]
    s_ref[...] = s


def _mm1(x, wc1):
    bm = 2000
    hh = H1 // 2
    return pl.pallas_call(
        _mm1_body,
        grid=(N // bm,),
        in_specs=[
            pl.BlockSpec((bm, D_IN), lambda i: (i, 0)),
            pl.BlockSpec((D_IN, 3 * H1), lambda i: (0, 0)),
        ],
        out_specs=[
            pl.BlockSpec((bm, hh), lambda i: (i, 0)),
            pl.BlockSpec((bm, hh), lambda i: (i, 0)),
            pl.BlockSpec((bm, hh), lambda i: (i, 0)),
            pl.BlockSpec((bm, hh), lambda i: (i, 0)),
            pl.BlockSpec((bm, H1), lambda i: (i, 0)),
        ],
        out_shape=[jax.ShapeDtypeStruct((N, hh), jnp.float32)] * 4
        + [jax.ShapeDtypeStruct((N, H1), jnp.float32)],
    )(x, wc1)


def _mm2_body(p_ref, s_ref, w_ref, b0_ref, b1_ref, t_ref):
    p = jnp.concatenate([p_ref[0], p_ref[1]], axis=1)
    h1 = (p - s_ref[...]) * (1.0 / 3.0)
    acc = jnp.dot(h1, w_ref[...], preferred_element_type=jnp.float32)
    s2 = acc[:, 0:H2]
    s3 = acc[:, 3 * H2:4 * H2]
    b0_ref[...] = jnp.concatenate(
        [s2 + acc[:, H2:2 * H2], s3 + acc[:, 4 * H2:5 * H2]], axis=1)
    b1_ref[...] = jnp.concatenate(
        [s2 + acc[:, 2 * H2:3 * H2], s3 + acc[:, 5 * H2:6 * H2]], axis=1)
    t_ref[...] = jnp.concatenate([s2, s3], axis=1)


def _mm2(parts, s, wc2):
    bm = 2000
    hh = H1 // 2
    return pl.pallas_call(
        _mm2_body,
        grid=(N // bm,),
        in_specs=[
            pl.BlockSpec((2, bm, hh), lambda i: (0, i, 0)),
            pl.BlockSpec((bm, H1), lambda i: (i, 0)),
            pl.BlockSpec((H1, 6 * H2), lambda i: (0, 0)),
        ],
        out_specs=[
            pl.BlockSpec((bm, 2 * H2), lambda i: (i, 0)),
            pl.BlockSpec((bm, 2 * H2), lambda i: (i, 0)),
            pl.BlockSpec((bm, 2 * H2), lambda i: (i, 0)),
        ],
        out_shape=[jax.ShapeDtypeStruct((N, 2 * H2), jnp.float32)] * 3,
    )(parts, s, wc2)


def _head_body(parts_ref, t_ref, eps_ref, mu_ref, lv_ref, z_ref):
    q = (parts_ref[0] + parts_ref[1] - t_ref[...]) * (1.0 / 3.0)
    mu = q[:, 0:H2]
    lv = q[:, H2:2 * H2]
    mu_ref[...] = mu
    lv_ref[...] = lv
    z_ref[...] = eps_ref[...] * jnp.exp(lv) + mu


def _head(parts, t, eps):
    bm = 2000
    return pl.pallas_call(
        _head_body,
        grid=(N // bm,),
        in_specs=[
            pl.BlockSpec((2, bm, 2 * H2), lambda i: (0, i, 0)),
            pl.BlockSpec((bm, 2 * H2), lambda i: (i, 0)),
            pl.BlockSpec((bm, H2), lambda i: (i, 0)),
        ],
        out_specs=[
            pl.BlockSpec((bm, H2), lambda i: (i, 0)),
            pl.BlockSpec((bm, H2), lambda i: (i, 0)),
            pl.BlockSpec((bm, H2), lambda i: (i, 0)),
        ],
        out_shape=[jax.ShapeDtypeStruct((N, H2), jnp.float32)] * 3,
    )(parts, t, eps)


def _dec_body(zr_ref, zc_ref, out_ref):
    out_ref[...] = lax.dot_general(
        zr_ref[...], zc_ref[...], (((1,), (1,)), ((), ())),
        preferred_element_type=jnp.float32)


def _dec(z):
    bm, bn = 1024, 2048
    return pl.pallas_call(
        _dec_body,
        grid=(pl.cdiv(N, bm), pl.cdiv(N, bn)),
        in_specs=[
            pl.BlockSpec((bm, H2), lambda i, j: (i, 0)),
            pl.BlockSpec((bn, H2), lambda i, j: (j, 0)),
        ],
        out_specs=pl.BlockSpec((bm, bn), lambda i, j: (i, j)),
        out_shape=jax.ShapeDtypeStruct((N, N), jnp.float32),
    )(z, z)


_SC_PARAMS = pltpu.CompilerParams(use_tc_tiling_on_sc=False)


def _zero_acc(s, zbuf, acc, h):
    def zrow(i, _):
        for j in range(h // 16):
            zbuf[i, pl.ds(j * 16, 16)] = jnp.zeros((16,), jnp.float32)
        return 0
    lax.fori_loop(0, ZROWS, zrow, 0)

    @pl.when(s < NSUB - 1)
    def _():
        for i in range(OWN // ZROWS):
            pltpu.sync_copy(zbuf, acc.at[pl.ds(s * OWN + i * ZROWS, ZROWS)])
        rem = OWN % ZROWS
        if rem:
            pltpu.sync_copy(zbuf.at[pl.ds(0, rem)],
                            acc.at[pl.ds(s * OWN + OWN - rem, rem)])

    @pl.when(s == NSUB - 1)
    def _():
        for i in range(OWN_LAST // ZROWS):
            pltpu.sync_copy(zbuf, acc.at[pl.ds(s * OWN + i * ZROWS, ZROWS)])
        rem = OWN_LAST % ZROWS
        if rem:
            pltpu.sync_copy(zbuf.at[pl.ds(0, rem)],
                            acc.at[pl.ds(s * OWN + OWN_LAST - rem, rem)])
    plsc.subcore_barrier()


def _publish(c, s, acc, out_hbm):
    plsc.subcore_barrier()

    @pl.when(s < NSUB - 1)
    def _():
        pltpu.sync_copy(acc.at[pl.ds(s * OWN, OWN)],
                        out_hbm.at[c, pl.ds(s * OWN, OWN)])

    @pl.when(s == NSUB - 1)
    def _():
        pltpu.sync_copy(acc.at[pl.ds(s * OWN, OWN_LAST)],
                        out_hbm.at[c, pl.ds(s * OWN, OWN_LAST)])


def _make_edge_runner(h, nchunk, acc, colv, rowv, valv, gbuf, semg):
    def run(r_hbm, c_hbm, v_hbm, t_hbm, widx):
        pltpu.sync_copy(c_hbm.at[pl.ds(widx * nchunk, nchunk)], colv)
        pltpu.sync_copy(r_hbm.at[pl.ds(widx * nchunk, nchunk)], rowv)
        pltpu.sync_copy(v_hbm.at[pl.ds(widx * nchunk, nchunk)], valv)

        def issue(k, b):
            pltpu.async_copy(t_hbm.at[colv.at[k]], gbuf.at[b], semg.at[b])

        for b in range(NBUF):
            issue(b, b)
        outer = nchunk // NBUF

        def outer_body(g, _):
            for b in range(NBUF):
                k = g * NBUF + b
                pltpu.make_async_copy(
                    t_hbm.at[colv.at[k]], gbuf.at[b], semg.at[b]).wait()
                gb = gbuf.at[b]
                vrow = valv.at[k]

                @plsc.parallel_loop(0, CHUNK, step=16)
                def scale(e0):
                    grp = vrow[pl.ds(e0, 16)]
                    for l in range(16):
                        vv = lax.gather(
                            grp, jnp.full((16, 1), l, jnp.int32),
                            lax.GatherDimensionNumbers(
                                offset_dims=(), collapsed_slice_dims=(0,),
                                start_index_map=(0,)),
                            (1,),
                            mode=lax.GatherScatterMode.PROMISE_IN_BOUNDS)
                        for j in range(h // 16):
                            sl = pl.ds(j * 16, 16)
                            gb[e0 + l, sl] = gb[e0 + l, sl] * vv

                pltpu.sync_copy(gb, acc.at[rowv.at[k]], add=True)

                @pl.when(g < outer - 1)
                def _():
                    issue(k + NBUF, b)
            return 0
        lax.fori_loop(0, outer, outer_body, 0)
    return run


def _sc_scratch(h, nchunk):
    return [
        pltpu.VMEM((nchunk, CHUNK), jnp.int32),
        pltpu.VMEM((nchunk, CHUNK), jnp.int32),
        pltpu.VMEM((nchunk, CHUNK), jnp.float32),
        pltpu.VMEM((NBUF, CHUNK, h), jnp.float32),
        pltpu.VMEM((ZROWS, h), jnp.float32),
        pltpu.VMEM_SHARED((N, h), jnp.float32),
        pltpu.SemaphoreType.DMA((NBUF,)),
    ]


def _spmm_l1(t0l, t0h, t1l, t1h, r0, c0, v0, r1, c1, v1):
    h = H1 // 2
    mesh = plsc.VectorSubcoreMesh(core_axis_name="c", subcore_axis_name="s")

    @functools.partial(
        pl.kernel,
        out_type=jax.ShapeDtypeStruct((NCORES, N, h), jnp.float32),
        mesh=mesh,
        compiler_params=_SC_PARAMS,
        scratch_types=_sc_scratch(h, NCHUNK),
    )
    def spmm(t0l_hbm, t0h_hbm, t1l_hbm, t1h_hbm, r0_hbm, c0_hbm, v0_hbm,
             r1_hbm, c1_hbm, v1_hbm, out_hbm,
             colv, rowv, valv, gbuf, zbuf, acc, semg):
        c = lax.axis_index("c")
        s = lax.axis_index("s")
        _zero_acc(s, zbuf, acc, h)
        run = _make_edge_runner(h, NCHUNK, acc, colv, rowv, valv, gbuf, semg)

        @pl.when(c == 0)
        def _():
            run(r0_hbm, c0_hbm, v0_hbm, t0l_hbm, s)
            run(r1_hbm, c1_hbm, v1_hbm, t1l_hbm, s)

        @pl.when(c == 1)
        def _():
            run(r0_hbm, c0_hbm, v0_hbm, t0h_hbm, s)
            run(r1_hbm, c1_hbm, v1_hbm, t1h_hbm, s)

        _publish(c, s, acc, out_hbm)

    return spmm(t0l, t0h, t1l, t1h, r0, c0, v0, r1, c1, v1)


def _spmm_sc(h, t0, t1, r0, c0, v0, r1, c1, v1):
    mesh = plsc.VectorSubcoreMesh(core_axis_name="c", subcore_axis_name="s")

    @functools.partial(
        pl.kernel,
        out_type=jax.ShapeDtypeStruct((NCORES, N, h), jnp.float32),
        mesh=mesh,
        compiler_params=_SC_PARAMS,
        scratch_types=_sc_scratch(h, NCHUNK),
    )
    def spmm(t0_hbm, t1_hbm, r0_hbm, c0_hbm, v0_hbm, r1_hbm, c1_hbm, v1_hbm,
             out_hbm, colv, rowv, valv, gbuf, zbuf, acc, semg):
        c = lax.axis_index("c")
        s = lax.axis_index("s")
        _zero_acc(s, zbuf, acc, h)
        run = _make_edge_runner(h, NCHUNK, acc, colv, rowv, valv, gbuf, semg)

        @pl.when(c == 0)
        def _():
            run(r0_hbm, c0_hbm, v0_hbm, t0_hbm, s)

        @pl.when(c == 1)
        def _():
            run(r1_hbm, c1_hbm, v1_hbm, t1_hbm, s)

        _publish(c, s, acc, out_hbm)

    return spmm(t0, t1, r0, c0, v0, r1, c1, v1)

    return spmm(t0, t1, r0, c0, v0, r1, c1, v1)


def kernel(x, adj0_indices, adj0_values, adj1_indices, adj1_values,
           W1, W1_dc, W1_dd, W2, W2_dc, W2_dd, W3, W3_dc, W3_dd):
    wc1 = jnp.concatenate([W1, W1_dc, W1_dd], axis=1)
    wc2 = jnp.concatenate([W2, W2_dc, W2_dd, W3, W3_dc, W3_dd], axis=1)
    eps = jax.random.normal(jax.random.key(42), (N, H2), jnp.float32)
    r0 = adj0_indices[0].reshape(E // CHUNK, CHUNK)
    c0 = adj0_indices[1].reshape(E // CHUNK, CHUNK)
    r1 = adj1_indices[0].reshape(E // CHUNK, CHUNK)
    c1 = adj1_indices[1].reshape(E // CHUNK, CHUNK)
    v0 = adj0_values.reshape(E // CHUNK, CHUNK)
    v1 = adj1_values.reshape(E // CHUNK, CHUNK)

    a0l, a0h, a1l, a1h, s = _mm1(x, wc1)
    parts1 = _spmm_l1(a0l, a0h, a1l, a1h, r0, c0, v0, r1, c1, v1)
    b0, b1, t = _mm2(parts1, s, wc2)
    parts2 = _spmm_sc(2 * H2, b0, b1, r0, c0, v0, r1, c1, v1)
    mu, logvar, z = _head(parts2, t, eps)
    adj_rec = _dec(z)
    return (adj_rec, mu, logvar)


# speedup vs baseline: 1.0055x; 1.0055x over previous
import functools

import jax
import jax.numpy as jnp
from jax import lax
from jax.experimental import pallas as pl
from jax.experimental.pallas import tpu as pltpu
from jax.experimental.pallas import tpu_sc as plsc

N = 10000
E = 160000
D_IN, H1, H2 = 256, 128, 32

NCORES = 2
NSUB = 16
NWORK = NCORES * NSUB
EPW = E // NWORK
CHUNK = 80
NCHUNK = (E // NSUB) // CHUNK
NBUF = 5
OWN = 632
OWN_LAST = N - 15 * OWN
ZROWS = 40


def _mm1_body(x_ref, w_ref, a0l_ref, a0h_ref, a1l_ref, a1h_ref, s_ref):
    acc = jnp.dot(x_ref[...], w_ref[...], preferred_element_type=jnp.float32)
    hh = H1 // 2
    s = acc[:, 0:H1]
    a0 = s + acc[:, H1:2 * H1]
    a1 = s + acc[:, 2 * H1:3 * H1]
    a0l_ref[...] = a0[:, 0:hh]
    a0h_ref[...] = a0[:, hh:2 * hh]
    a1l_ref[...] = a1[:, 0:hh]
    a1h_ref[...] = a1[:, hh:2 * hh]
    s_ref[...] = s


def _mm1(x, wc1):
    bm = 2000
    hh = H1 // 2
    return pl.pallas_call(
        _mm1_body,
        grid=(N // bm,),
        in_specs=[
            pl.BlockSpec((bm, D_IN), lambda i: (i, 0)),
            pl.BlockSpec((D_IN, 3 * H1), lambda i: (0, 0)),
        ],
        out_specs=[
            pl.BlockSpec((bm, hh), lambda i: (i, 0)),
            pl.BlockSpec((bm, hh), lambda i: (i, 0)),
            pl.BlockSpec((bm, hh), lambda i: (i, 0)),
            pl.BlockSpec((bm, hh), lambda i: (i, 0)),
            pl.BlockSpec((bm, H1), lambda i: (i, 0)),
        ],
        out_shape=[jax.ShapeDtypeStruct((N, hh), jnp.float32)] * 4
        + [jax.ShapeDtypeStruct((N, H1), jnp.float32)],
    )(x, wc1)


def _mm2_body(p_ref, s_ref, w_ref, b0_ref, b1_ref, t_ref):
    p = jnp.concatenate([p_ref[0], p_ref[1]], axis=1)
    h1 = (p - s_ref[...]) * (1.0 / 3.0)
    acc = jnp.dot(h1, w_ref[...], preferred_element_type=jnp.float32)
    s2 = acc[:, 0:H2]
    s3 = acc[:, 3 * H2:4 * H2]
    b0_ref[...] = jnp.concatenate(
        [s2 + acc[:, H2:2 * H2], s3 + acc[:, 4 * H2:5 * H2]], axis=1)
    b1_ref[...] = jnp.concatenate(
        [s2 + acc[:, 2 * H2:3 * H2], s3 + acc[:, 5 * H2:6 * H2]], axis=1)
    t_ref[...] = jnp.concatenate([s2, s3], axis=1)


def _mm2(parts, s, wc2):
    bm = 2000
    hh = H1 // 2
    return pl.pallas_call(
        _mm2_body,
        grid=(N // bm,),
        in_specs=[
            pl.BlockSpec((2, bm, hh), lambda i: (0, i, 0)),
            pl.BlockSpec((bm, H1), lambda i: (i, 0)),
            pl.BlockSpec((H1, 6 * H2), lambda i: (0, 0)),
        ],
        out_specs=[
            pl.BlockSpec((bm, 2 * H2), lambda i: (i, 0)),
            pl.BlockSpec((bm, 2 * H2), lambda i: (i, 0)),
            pl.BlockSpec((bm, 2 * H2), lambda i: (i, 0)),
        ],
        out_shape=[jax.ShapeDtypeStruct((N, 2 * H2), jnp.float32)] * 3,
    )(parts, s, wc2)


def _head_body(parts_ref, t_ref, eps_ref, mu_ref, lv_ref, z_ref):
    q = (parts_ref[0] + parts_ref[1] - t_ref[...]) * (1.0 / 3.0)
    mu = q[:, 0:H2]
    lv = q[:, H2:2 * H2]
    mu_ref[...] = mu
    lv_ref[...] = lv
    z_ref[...] = eps_ref[...] * jnp.exp(lv) + mu


def _head(parts, t, eps):
    bm = 2000
    return pl.pallas_call(
        _head_body,
        grid=(N // bm,),
        in_specs=[
            pl.BlockSpec((2, bm, 2 * H2), lambda i: (0, i, 0)),
            pl.BlockSpec((bm, 2 * H2), lambda i: (i, 0)),
            pl.BlockSpec((bm, H2), lambda i: (i, 0)),
        ],
        out_specs=[
            pl.BlockSpec((bm, H2), lambda i: (i, 0)),
            pl.BlockSpec((bm, H2), lambda i: (i, 0)),
            pl.BlockSpec((bm, H2), lambda i: (i, 0)),
        ],
        out_shape=[jax.ShapeDtypeStruct((N, H2), jnp.float32)] * 3,
    )(parts, t, eps)


def _dec_body(zr_ref, zc_ref, out_ref):
    out_ref[...] = lax.dot_general(
        zr_ref[...], zc_ref[...], (((1,), (1,)), ((), ())),
        preferred_element_type=jnp.float32)


def _dec(z):
    bm, bn = 1024, 2048
    return pl.pallas_call(
        _dec_body,
        grid=(pl.cdiv(N, bm), pl.cdiv(N, bn)),
        in_specs=[
            pl.BlockSpec((bm, H2), lambda i, j: (i, 0)),
            pl.BlockSpec((bn, H2), lambda i, j: (j, 0)),
        ],
        out_specs=pl.BlockSpec((bm, bn), lambda i, j: (i, j)),
        out_shape=jax.ShapeDtypeStruct((N, N), jnp.float32),
    )(z, z)


_SC_PARAMS = pltpu.CompilerParams(use_tc_tiling_on_sc=False)


def _zero_acc(s, zbuf, acc, h):
    def zrow(i, _):
        for j in range(h // 16):
            zbuf[i, pl.ds(j * 16, 16)] = jnp.zeros((16,), jnp.float32)
        return 0
    lax.fori_loop(0, ZROWS, zrow, 0)

    @pl.when(s < NSUB - 1)
    def _():
        for i in range(OWN // ZROWS):
            pltpu.sync_copy(zbuf, acc.at[pl.ds(s * OWN + i * ZROWS, ZROWS)])
        rem = OWN % ZROWS
        if rem:
            pltpu.sync_copy(zbuf.at[pl.ds(0, rem)],
                            acc.at[pl.ds(s * OWN + OWN - rem, rem)])

    @pl.when(s == NSUB - 1)
    def _():
        for i in range(OWN_LAST // ZROWS):
            pltpu.sync_copy(zbuf, acc.at[pl.ds(s * OWN + i * ZROWS, ZROWS)])
        rem = OWN_LAST % ZROWS
        if rem:
            pltpu.sync_copy(zbuf.at[pl.ds(0, rem)],
                            acc.at[pl.ds(s * OWN + OWN_LAST - rem, rem)])
    plsc.subcore_barrier()


def _publish(c, s, acc, out_hbm):
    plsc.subcore_barrier()

    @pl.when(s < NSUB - 1)
    def _():
        pltpu.sync_copy(acc.at[pl.ds(s * OWN, OWN)],
                        out_hbm.at[c, pl.ds(s * OWN, OWN)])

    @pl.when(s == NSUB - 1)
    def _():
        pltpu.sync_copy(acc.at[pl.ds(s * OWN, OWN_LAST)],
                        out_hbm.at[c, pl.ds(s * OWN, OWN_LAST)])


def _make_edge_runner(h, nchunk, acc, colv, rowv, valv, gbuf, semg):
    def run(r_hbm, c_hbm, v_hbm, t_hbm, widx):
        pltpu.sync_copy(c_hbm.at[pl.ds(widx * nchunk, nchunk)], colv)
        pltpu.sync_copy(r_hbm.at[pl.ds(widx * nchunk, nchunk)], rowv)
        pltpu.sync_copy(v_hbm.at[pl.ds(widx * nchunk, nchunk)], valv)

        def issue(k, b):
            pltpu.async_copy(t_hbm.at[colv.at[k]], gbuf.at[b], semg.at[b])

        for b in range(NBUF):
            issue(b, b)
        outer = nchunk // NBUF

        def outer_body(g, _):
            for b in range(NBUF):
                k = g * NBUF + b
                pltpu.make_async_copy(
                    t_hbm.at[colv.at[k]], gbuf.at[b], semg.at[b]).wait()
                gb = gbuf.at[b]
                vrow = valv.at[k]

                @plsc.parallel_loop(0, CHUNK, unroll=4)
                def scale(e):
                    g16 = (e // 16) * 16
                    grp = vrow[pl.ds(g16, 16)]
                    vv = lax.gather(
                        grp, jnp.broadcast_to(e - g16, (16, 1)),
                        lax.GatherDimensionNumbers(
                            offset_dims=(), collapsed_slice_dims=(0,),
                            start_index_map=(0,)),
                        (1,), mode=lax.GatherScatterMode.PROMISE_IN_BOUNDS)
                    for j in range(h // 16):
                        sl = pl.ds(j * 16, 16)
                        gb[e, sl] = gb[e, sl] * vv

                pltpu.sync_copy(gb, acc.at[rowv.at[k]], add=True)

                @pl.when(g < outer - 1)
                def _():
                    issue(k + NBUF, b)
            return 0
        lax.fori_loop(0, outer, outer_body, 0)
    return run


def _sc_scratch(h, nchunk):
    return [
        pltpu.VMEM((nchunk, CHUNK), jnp.int32),
        pltpu.VMEM((nchunk, CHUNK), jnp.int32),
        pltpu.VMEM((nchunk, CHUNK), jnp.float32),
        pltpu.VMEM((NBUF, CHUNK, h), jnp.float32),
        pltpu.VMEM((ZROWS, h), jnp.float32),
        pltpu.VMEM_SHARED((N, h), jnp.float32),
        pltpu.SemaphoreType.DMA((NBUF,)),
    ]


def _spmm_l1(t0l, t0h, t1l, t1h, r0, c0, v0, r1, c1, v1):
    h = H1 // 2
    mesh = plsc.VectorSubcoreMesh(core_axis_name="c", subcore_axis_name="s")

    @functools.partial(
        pl.kernel,
        out_type=jax.ShapeDtypeStruct((NCORES, N, h), jnp.float32),
        mesh=mesh,
        compiler_params=_SC_PARAMS,
        scratch_types=_sc_scratch(h, NCHUNK),
    )
    def spmm(t0l_hbm, t0h_hbm, t1l_hbm, t1h_hbm, r0_hbm, c0_hbm, v0_hbm,
             r1_hbm, c1_hbm, v1_hbm, out_hbm,
             colv, rowv, valv, gbuf, zbuf, acc, semg):
        c = lax.axis_index("c")
        s = lax.axis_index("s")
        _zero_acc(s, zbuf, acc, h)
        run = _make_edge_runner(h, NCHUNK, acc, colv, rowv, valv, gbuf, semg)

        @pl.when(c == 0)
        def _():
            run(r0_hbm, c0_hbm, v0_hbm, t0l_hbm, s)
            run(r1_hbm, c1_hbm, v1_hbm, t1l_hbm, s)

        @pl.when(c == 1)
        def _():
            run(r0_hbm, c0_hbm, v0_hbm, t0h_hbm, s)
            run(r1_hbm, c1_hbm, v1_hbm, t1h_hbm, s)

        _publish(c, s, acc, out_hbm)

    return spmm(t0l, t0h, t1l, t1h, r0, c0, v0, r1, c1, v1)


def _spmm_sc(h, t0, t1, r0, c0, v0, r1, c1, v1):
    mesh = plsc.VectorSubcoreMesh(core_axis_name="c", subcore_axis_name="s")

    @functools.partial(
        pl.kernel,
        out_type=jax.ShapeDtypeStruct((NCORES, N, h), jnp.float32),
        mesh=mesh,
        compiler_params=_SC_PARAMS,
        scratch_types=_sc_scratch(h, NCHUNK),
    )
    def spmm(t0_hbm, t1_hbm, r0_hbm, c0_hbm, v0_hbm, r1_hbm, c1_hbm, v1_hbm,
             out_hbm, colv, rowv, valv, gbuf, zbuf, acc, semg):
        c = lax.axis_index("c")
        s = lax.axis_index("s")
        _zero_acc(s, zbuf, acc, h)
        run = _make_edge_runner(h, NCHUNK, acc, colv, rowv, valv, gbuf, semg)

        @pl.when(c == 0)
        def _():
            run(r0_hbm, c0_hbm, v0_hbm, t0_hbm, s)

        @pl.when(c == 1)
        def _():
            run(r1_hbm, c1_hbm, v1_hbm, t1_hbm, s)

        _publish(c, s, acc, out_hbm)

    return spmm(t0, t1, r0, c0, v0, r1, c1, v1)

    return spmm(t0, t1, r0, c0, v0, r1, c1, v1)


def kernel(x, adj0_indices, adj0_values, adj1_indices, adj1_values,
           W1, W1_dc, W1_dd, W2, W2_dc, W2_dd, W3, W3_dc, W3_dd):
    wc1 = jnp.concatenate([W1, W1_dc, W1_dd], axis=1)
    wc2 = jnp.concatenate([W2, W2_dc, W2_dd, W3, W3_dc, W3_dd], axis=1)
    eps = jax.random.normal(jax.random.key(42), (N, H2), jnp.float32)
    r0 = adj0_indices[0].reshape(E // CHUNK, CHUNK)
    c0 = adj0_indices[1].reshape(E // CHUNK, CHUNK)
    r1 = adj1_indices[0].reshape(E // CHUNK, CHUNK)
    c1 = adj1_indices[1].reshape(E // CHUNK, CHUNK)
    v0 = adj0_values.reshape(E // CHUNK, CHUNK)
    v1 = adj1_values.reshape(E // CHUNK, CHUNK)

    a0l, a0h, a1l, a1h, s = _mm1(x, wc1)
    parts1 = _spmm_l1(a0l, a0h, a1l, a1h, r0, c0, v0, r1, c1, v1)
    b0, b1, t = _mm2(parts1, s, wc2)
    parts2 = _spmm_sc(2 * H2, b0, b1, r0, c0, v0, r1, c1, v1)
    mu, logvar, z = _head(parts2, t, eps)
    adj_rec = _dec(z)
    return (adj_rec, mu, logvar)
